# baseline (speedup 1.0000x reference)
import functools

import jax
import jax.numpy as jnp
from jax import lax
from jax.experimental import pallas as pl
from jax.experimental.pallas import tpu as pltpu
from jax.experimental.pallas import tpu_sc as plsc

VOCAB = 100000
D = 128
B = 16384
L = 200
BL = B * L

NC = 2
NS = 16
NW = NC * NS
CH = 128
SUP = 16
SUP_ROWS = CH * SUP
PER_W = BL // NW
N_SUP = PER_W // SUP_ROWS
CHUNK_ROWS_PER_W = PER_W // CH


def _sc_gather_body(cw, pw, nw, in_t, out_t, oc, op, on,
                    idx_v, b0, b1, b2, b3, semg, semw0, semw1):
    wid = lax.axis_index("s") * NC + lax.axis_index("c")
    base_crow = wid * CHUNK_ROWS_PER_W
    banks = ((b0, b1, semw0), (b2, b3, semw1))
    for idx_hbm, table, out_hbm in ((cw, in_t, oc), (pw, out_t, op), (nw, out_t, on)):
        def super_body(j, carry, idx_hbm=idx_hbm, table=table, out_hbm=out_hbm):
            crow0 = base_crow + j * SUP
            row0 = crow0 * CH
            pltpu.sync_copy(idx_hbm.at[pl.ds(crow0, SUP), :], idx_v)
            live_wh = {}
            for g in range(SUP // 2):
                bank = g % 2
                bufa, bufb, semw = banks[bank]
                if g >= 2:
                    for h in live_wh[bank]:
                        h.wait()
                else:
                    @pl.when(j > 0)
                    def _drain_prev_super(bufa=bufa, bufb=bufb, semw=semw, out_hbm=out_hbm):
                        pltpu.make_async_copy(bufa, out_hbm.at[pl.ds(0, CH)], semw).wait()
                        pltpu.make_async_copy(bufb, out_hbm.at[pl.ds(0, CH)], semw).wait()
                c0 = g * 2
                gh = [
                    pltpu.async_copy(table.at[idx_v.at[c0]], bufa, semg),
                    pltpu.async_copy(table.at[idx_v.at[c0 + 1]], bufb, semg),
                ]
                for h in gh:
                    h.wait()
                live_wh[bank] = [
                    pltpu.async_copy(bufa, out_hbm.at[pl.ds(row0 + c0 * CH, CH)], semw),
                    pltpu.async_copy(bufb, out_hbm.at[pl.ds(row0 + (c0 + 1) * CH, CH)], semw),
                ]
            return carry
        lax.fori_loop(0, N_SUP, super_body, 0)
        for bufa, bufb, semw in banks:
            pltpu.make_async_copy(bufa, out_hbm.at[pl.ds(0, CH)], semw).wait()
            pltpu.make_async_copy(bufb, out_hbm.at[pl.ds(0, CH)], semw).wait()


_sc_gather = functools.partial(
    pl.kernel,
    mesh=plsc.VectorSubcoreMesh(core_axis_name="c", subcore_axis_name="s"),
    compiler_params=pltpu.CompilerParams(use_tc_tiling_on_sc=False),
    out_type=[jax.ShapeDtypeStruct((BL, D // 2), jnp.int32)] * 3,
    scratch_types=[
        pltpu.VMEM((SUP, CH), jnp.int32),
        pltpu.VMEM((CH, D // 2), jnp.int32),
        pltpu.VMEM((CH, D // 2), jnp.int32),
        pltpu.VMEM((CH, D // 2), jnp.int32),
        pltpu.VMEM((CH, D // 2), jnp.int32),
        pltpu.SemaphoreType.DMA,
        pltpu.SemaphoreType.DMA,
        pltpu.SemaphoreType.DMA,
    ],
)(_sc_gather_body)


G = 8
NG = B // G

LOG2E = 1.4426950408889634
LN2 = 0.6931471805599453
INV = 0.5 / LN2


def _halves(x):
    lo = lax.bitcast_convert_type(lax.shift_left(x, 16), jnp.float32)
    hi = lax.bitcast_convert_type(
        lax.bitwise_and(x, jnp.int32(-65536)), jnp.float32)
    return lo, hi


def _tc_loss_body(c_ref, p_ref, n_ref, out_ref):
    g = pl.program_id(0)

    @pl.when(g == 0)
    def _init():
        out_ref[...] = jnp.zeros((1, 1), jnp.float32)

    total = jnp.float32(0.0)
    for b in range(G):
        cl, ch = _halves(c_ref[b * L:(b + 1) * L, :])
        pl_, ph = _halves(p_ref[b * L:(b + 1) * L, :])
        nl, nh = _halves(n_ref[b * L:(b + 1) * L, :])
        dn = (((1,), (1,)), ((), ()))
        ps = (lax.dot_general(cl, pl_, dn, preferred_element_type=jnp.float32)
              + lax.dot_general(ch, ph, dn, preferred_element_type=jnp.float32))
        ns = (lax.dot_general(cl, nl, dn, preferred_element_type=jnp.float32)
              + lax.dot_general(ch, nh, dn, preferred_element_type=jnp.float32))
        ap = jnp.abs(ps)
        an = jnp.abs(ns)
        lp = jnp.log(1.0 + jnp.exp(-ap))
        ln_ = jnp.log(1.0 + jnp.exp(-an))
        term = (lp + ln_) + ((ap - ps) + (an + ns)) * 0.5
        total = total + jnp.sum(term)
    out_ref[...] += jnp.full((1, 1), total, jnp.float32)

    @pl.when(g == NG - 1)
    def _finalize():
        out_ref[...] = out_ref[...] * (1.0 / float(BL))


def _tc_loss(oc, op, on):
    return pl.pallas_call(
        _tc_loss_body,
        grid=(NG,),
        in_specs=[pl.BlockSpec((G * L, D // 2), lambda i: (i, 0))] * 3,
        out_specs=pl.BlockSpec((1, 1), lambda i: (0, 0)),
        out_shape=jax.ShapeDtypeStruct((1, 1), jnp.float32),
    )(oc, op, on)


def kernel(center_word, pos_word, neg_word, in_emb, out_emb):
    cw = center_word.reshape(BL // CH, CH)
    pw = pos_word.reshape(BL // CH, CH)
    nw = neg_word.reshape(BL // CH, CH)
    ini = lax.bitcast_convert_type(
        in_emb.astype(jnp.bfloat16).reshape(VOCAB, D // 2, 2), jnp.int32)
    outi = lax.bitcast_convert_type(
        out_emb.astype(jnp.bfloat16).reshape(VOCAB, D // 2, 2), jnp.int32)
    oc, op, on = _sc_gather(cw, pw, nw, ini, outi)
    loss = _tc_loss(oc, op, on)
    return loss[0, 0]


# speedup vs baseline: 1.4796x; 1.4796x over previous
import functools

import jax
import jax.numpy as jnp
from jax import lax
from jax.experimental import pallas as pl
from jax.experimental.pallas import tpu as pltpu
from jax.experimental.pallas import tpu_sc as plsc

VOCAB = 100000
D = 128
B = 16384
L = 200
BL = B * L

NC = 2
NS = 16
NW = NC * NS
CH = 128
SUP = 16
SUP_ROWS = CH * SUP
PER_W = BL // NW
N_SUP = PER_W // SUP_ROWS
CHUNK_ROWS_PER_W = PER_W // CH


def _sc_gather_body(cw, pw, nw, in_t, out_t, oc, op, on,
                    idx_v, b0, b1, b2, b3, semg, semw0, semw1):
    wid = lax.axis_index("s") * NC + lax.axis_index("c")
    base_crow = wid * CHUNK_ROWS_PER_W
    banks = ((b0, b1, semw0), (b2, b3, semw1))
    for idx_hbm, table, out_hbm in ((cw, in_t, oc), (pw, out_t, op), (nw, out_t, on)):
        def super_body(j, carry, idx_hbm=idx_hbm, table=table, out_hbm=out_hbm):
            crow0 = base_crow + j * SUP
            row0 = crow0 * CH
            pltpu.sync_copy(idx_hbm.at[pl.ds(crow0, SUP), :], idx_v)
            live_wh = {}
            for g in range(SUP // 2):
                bank = g % 2
                bufa, bufb, semw = banks[bank]
                if g >= 2:
                    for h in live_wh[bank]:
                        h.wait()
                else:
                    @pl.when(j > 0)
                    def _drain_prev_super(bufa=bufa, bufb=bufb, semw=semw, out_hbm=out_hbm):
                        pltpu.make_async_copy(bufa, out_hbm.at[pl.ds(0, CH)], semw).wait()
                        pltpu.make_async_copy(bufb, out_hbm.at[pl.ds(0, CH)], semw).wait()
                c0 = g * 2
                gh = [
                    pltpu.async_copy(table.at[idx_v.at[c0]], bufa, semg),
                    pltpu.async_copy(table.at[idx_v.at[c0 + 1]], bufb, semg),
                ]
                for h in gh:
                    h.wait()
                live_wh[bank] = [
                    pltpu.async_copy(bufa, out_hbm.at[pl.ds(row0 + c0 * CH, CH)], semw),
                    pltpu.async_copy(bufb, out_hbm.at[pl.ds(row0 + (c0 + 1) * CH, CH)], semw),
                ]
            return carry
        lax.fori_loop(0, N_SUP, super_body, 0)
        for bufa, bufb, semw in banks:
            pltpu.make_async_copy(bufa, out_hbm.at[pl.ds(0, CH)], semw).wait()
            pltpu.make_async_copy(bufb, out_hbm.at[pl.ds(0, CH)], semw).wait()


_sc_gather = functools.partial(
    pl.kernel,
    mesh=plsc.VectorSubcoreMesh(core_axis_name="c", subcore_axis_name="s"),
    out_type=[jax.ShapeDtypeStruct((BL, D), jnp.float32)] * 3,
    scratch_types=[
        pltpu.VMEM((SUP, CH), jnp.int32),
        pltpu.VMEM((CH, D), jnp.float32),
        pltpu.VMEM((CH, D), jnp.float32),
        pltpu.VMEM((CH, D), jnp.float32),
        pltpu.VMEM((CH, D), jnp.float32),
        pltpu.SemaphoreType.DMA,
        pltpu.SemaphoreType.DMA,
        pltpu.SemaphoreType.DMA,
    ],
)(_sc_gather_body)


G = 8
NG = B // G

LOG2E = 1.4426950408889634
LN2 = 0.6931471805599453
INV = 0.5 / LN2


def _tc_loss_body(c_ref, p_ref, n_ref, out_ref):
    g = pl.program_id(0)

    @pl.when(g == 0)
    def _init():
        out_ref[...] = jnp.zeros((1, 1), jnp.float32)

    total = jnp.float32(0.0)
    for b in range(G):
        c = c_ref[b * L:(b + 1) * L, :]
        p = p_ref[b * L:(b + 1) * L, :]
        n = n_ref[b * L:(b + 1) * L, :]
        dn = (((1,), (1,)), ((), ()))
        ps = lax.dot_general(c, p, dn, preferred_element_type=jnp.float32)
        ns = lax.dot_general(c, n, dn, preferred_element_type=jnp.float32)
        ap = jnp.abs(ps)
        an = jnp.abs(ns)
        lp = jnp.log(1.0 + jnp.exp(-ap))
        ln_ = jnp.log(1.0 + jnp.exp(-an))
        term = (lp + ln_) + ((ap - ps) + (an + ns)) * 0.5
        total = total + jnp.sum(term)
    out_ref[...] += jnp.full((1, 1), total, jnp.float32)

    @pl.when(g == NG - 1)
    def _finalize():
        out_ref[...] = out_ref[...] * (1.0 / float(BL))


def _tc_loss(oc, op, on):
    return pl.pallas_call(
        _tc_loss_body,
        grid=(NG,),
        in_specs=[pl.BlockSpec((G * L, D), lambda i: (i, 0))] * 3,
        out_specs=pl.BlockSpec((1, 1), lambda i: (0, 0)),
        out_shape=jax.ShapeDtypeStruct((1, 1), jnp.float32),
    )(oc, op, on)


def kernel(center_word, pos_word, neg_word, in_emb, out_emb):
    cw = center_word.reshape(BL // CH, CH)
    pw = pos_word.reshape(BL // CH, CH)
    nw = neg_word.reshape(BL // CH, CH)
    oc, op, on = _sc_gather(cw, pw, nw, in_emb, out_emb)
    loss = _tc_loss(oc, op, on)
    return loss[0, 0]


# speedup vs baseline: 1.7723x; 1.1979x over previous
import functools

import jax
import jax.numpy as jnp
from jax import lax
from jax.experimental import pallas as pl
from jax.experimental.pallas import tpu as pltpu
from jax.experimental.pallas import tpu_sc as plsc

VOCAB = 100000
D = 128
B = 16384
L = 200
BL = B * L

CHUNKS = 4
BLC = BL // CHUNKS
NC = 2
NS = 16
NW = NC * NS
CH = 128
SUP = 8
PER_W = BLC // NW
N_SUP = PER_W // (CH * SUP)
CHUNK_ROWS_PER_W = PER_W // CH


def _sc_gather_body(cw, pw, nw, in_t, out_t, oc, op, on,
                    idx_v, b0, b1, b2, b3, semg, semw0, semw1):
    wid = lax.axis_index("s") * NC + lax.axis_index("c")
    base_crow = wid * CHUNK_ROWS_PER_W
    banks = ((b0, b1, semw0), (b2, b3, semw1))
    for idx_hbm, table, out_hbm in ((cw, in_t, oc), (pw, out_t, op), (nw, out_t, on)):
        def super_body(j, carry, idx_hbm=idx_hbm, table=table, out_hbm=out_hbm):
            crow0 = base_crow + j * SUP
            row0 = crow0 * CH
            pltpu.sync_copy(idx_hbm.at[pl.ds(crow0, SUP), :], idx_v)
            live_wh = {}
            for g in range(SUP // 2):
                bank = g % 2
                bufa, bufb, semw = banks[bank]
                if g >= 2:
                    for h in live_wh[bank]:
                        h.wait()
                else:
                    @pl.when(j > 0)
                    def _drain_prev_super(bufa=bufa, bufb=bufb, semw=semw, out_hbm=out_hbm):
                        pltpu.make_async_copy(bufa, out_hbm.at[pl.ds(0, CH)], semw).wait()
                        pltpu.make_async_copy(bufb, out_hbm.at[pl.ds(0, CH)], semw).wait()
                c0 = g * 2
                gh = [
                    pltpu.async_copy(table.at[idx_v.at[c0]], bufa, semg),
                    pltpu.async_copy(table.at[idx_v.at[c0 + 1]], bufb, semg),
                ]
                for h in gh:
                    h.wait()
                live_wh[bank] = [
                    pltpu.async_copy(bufa, out_hbm.at[pl.ds(row0 + c0 * CH, CH)], semw),
                    pltpu.async_copy(bufb, out_hbm.at[pl.ds(row0 + (c0 + 1) * CH, CH)], semw),
                ]
            return carry
        lax.fori_loop(0, N_SUP, super_body, 0)
        for bufa, bufb, semw in banks:
            pltpu.make_async_copy(bufa, out_hbm.at[pl.ds(0, CH)], semw).wait()
            pltpu.make_async_copy(bufb, out_hbm.at[pl.ds(0, CH)], semw).wait()


_sc_gather = functools.partial(
    pl.kernel,
    mesh=plsc.VectorSubcoreMesh(core_axis_name="c", subcore_axis_name="s"),
    out_type=[jax.ShapeDtypeStruct((BLC, D), jnp.float32)] * 3,
    scratch_types=[
        pltpu.VMEM((SUP, CH), jnp.int32),
        pltpu.VMEM((CH, D), jnp.float32),
        pltpu.VMEM((CH, D), jnp.float32),
        pltpu.VMEM((CH, D), jnp.float32),
        pltpu.VMEM((CH, D), jnp.float32),
        pltpu.SemaphoreType.DMA,
        pltpu.SemaphoreType.DMA,
        pltpu.SemaphoreType.DMA,
    ],
)(_sc_gather_body)


G = 8
NG = BLC // (G * L)

LOG2E = 1.4426950408889634
LN2 = 0.6931471805599453
INV = 0.5 / LN2


def _tc_loss_body(c_ref, p_ref, n_ref, out_ref):
    g = pl.program_id(0)

    @pl.when(g == 0)
    def _init():
        out_ref[...] = jnp.zeros((1, 1), jnp.float32)

    total = jnp.float32(0.0)
    for b in range(G):
        c = c_ref[b * L:(b + 1) * L, :]
        p = p_ref[b * L:(b + 1) * L, :]
        n = n_ref[b * L:(b + 1) * L, :]
        dn = (((1,), (1,)), ((), ()))
        ps = lax.dot_general(c, p, dn, preferred_element_type=jnp.float32)
        ns = lax.dot_general(c, n, dn, preferred_element_type=jnp.float32)
        ap = jnp.abs(ps)
        an = jnp.abs(ns)
        lp = jnp.log(1.0 + jnp.exp(-ap))
        ln_ = jnp.log(1.0 + jnp.exp(-an))
        term = (lp + ln_) + ((ap - ps) + (an + ns)) * 0.5
        total = total + jnp.sum(term)
    out_ref[...] += jnp.full((1, 1), total, jnp.float32)


def _tc_loss(oc, op, on):
    return pl.pallas_call(
        _tc_loss_body,
        grid=(NG,),
        in_specs=[pl.BlockSpec((G * L, D), lambda i: (i, 0))] * 3,
        out_specs=pl.BlockSpec((1, 1), lambda i: (0, 0)),
        out_shape=jax.ShapeDtypeStruct((1, 1), jnp.float32),
    )(oc, op, on)


def kernel(center_word, pos_word, neg_word, in_emb, out_emb):
    cw = center_word.reshape(BL // CH, CH)
    pw = pos_word.reshape(BL // CH, CH)
    nw = neg_word.reshape(BL // CH, CH)
    rows = BLC // CH
    partials = []
    for k in range(CHUNKS):
        sl = slice(k * rows, (k + 1) * rows)
        oc, op, on = _sc_gather(cw[sl], pw[sl], nw[sl], in_emb, out_emb)
        partials.append(_tc_loss(oc, op, on))
    total = sum(p[0, 0] for p in partials)
    return total * (1.0 / float(BL))
